# Initial kernel scaffold; baseline (speedup 1.0000x reference)
import jax
import jax.numpy as jnp
from jax.experimental import pallas as pl


def kernel(data, coeff_cache):
    raise NotImplementedError("write your pallas kernel here")


# speedup vs baseline: 18.9310x; 18.9310x over previous
import jax
import jax.numpy as jnp
import numpy as np
from jax.experimental import pallas as pl
from jax.experimental.pallas import tpu as pltpu

_NUM_SUMS = 3
_BVR_LEN = 64
_NUM_CACHE = 128
_P = 2 ** _NUM_SUMS
_LB = 128

_COMBS = ((np.arange(_P)[:, None] >> np.arange(_NUM_SUMS - 1, -1, -1)[None, :]) & 1)


def _sbvr_kernel(p_smem, x_ref, out_ref, n2p_smem, psq_smem):
    @pl.when(pl.program_id(0) == 0)
    def _init():
        def body(c, _):
            for j in range(_P):
                p = p_smem[c, j]
                n2p_smem[c, j] = -2.0 * p
                psq_smem[c, j] = p * p
            return 0
        jax.lax.fori_loop(0, _NUM_CACHE, body, 0)

    x = x_ref[...]
    x2 = x * x

    inf = jnp.full((1, _LB), jnp.inf, dtype=jnp.float32)
    zero_row = jnp.zeros((1, _LB), dtype=jnp.float32)
    init = (inf, jnp.zeros((1, _LB), dtype=jnp.int32)) + tuple(
        zero_row for _ in range(_P))

    def c_body(c, carry):
        best_mse, best_idx = carry[0], carry[1]
        sps = carry[2:]
        md = None
        for j in range(_P):
            z = x * n2p_smem[c, j] + psq_smem[c, j]
            md = z if md is None else jnp.minimum(md, z)
        md = x2 + md
        mse_c = jnp.sum(md, axis=0, keepdims=True)
        upd = mse_c < best_mse
        best_mse = jnp.where(upd, mse_c, best_mse)
        best_idx = jnp.where(upd, c, best_idx)
        new_sps = tuple(
            jnp.where(upd, p_smem[c, j], sps[j]) for j in range(_P))
        return (best_mse, best_idx) + new_sps

    carry = jax.lax.fori_loop(0, _NUM_CACHE, c_body, init)
    sps = carry[2:]

    q = None
    dbest = None
    for j in range(_P):
        spb = jnp.broadcast_to(sps[j], (_BVR_LEN, _LB))
        t = x - spb
        d = t * t
        if j == 0:
            q, dbest = spb, d
        else:
            upd = d < dbest
            dbest = jnp.where(upd, d, dbest)
            q = jnp.where(upd, spb, q)
    out_ref[...] = q


def kernel(data, coeff_cache):
    orig_shape = data.shape
    flat = data.reshape(-1)
    g = flat.shape[0] // _BVR_LEN
    xt = flat.reshape(g, _BVR_LEN).T

    bin_t = jnp.asarray(_COMBS.astype(np.float32)).T
    points = coeff_cache @ bin_t

    out_t = pl.pallas_call(
        _sbvr_kernel,
        grid=(g // _LB,),
        in_specs=[
            pl.BlockSpec(memory_space=pltpu.SMEM),
            pl.BlockSpec((_BVR_LEN, _LB), lambda i: (0, i)),
        ],
        out_specs=pl.BlockSpec((_BVR_LEN, _LB), lambda i: (0, i)),
        out_shape=jax.ShapeDtypeStruct((_BVR_LEN, g), jnp.float32),
        scratch_shapes=[
            pltpu.SMEM((_NUM_CACHE, _P), jnp.float32),
            pltpu.SMEM((_NUM_CACHE, _P), jnp.float32),
        ],
    )(points, xt)

    return out_t.T.reshape(orig_shape)


# speedup vs baseline: 24.9795x; 1.3195x over previous
import jax
import jax.numpy as jnp
import numpy as np
from jax.experimental import pallas as pl
from jax.experimental.pallas import tpu as pltpu

_NUM_SUMS = 3
_BVR_LEN = 64
_NUM_CACHE = 128
_P = 2 ** _NUM_SUMS
_LB = 128

_COMBS = ((np.arange(_P)[:, None] >> np.arange(_NUM_SUMS - 1, -1, -1)[None, :]) & 1)


def _sbvr_kernel(p_smem, x_ref, out_ref, n2p_smem, psq_smem):
    @pl.when(pl.program_id(0) == 0)
    def _init():
        def body(c, _):
            for j in range(_P):
                p = p_smem[c, j]
                n2p_smem[c, j] = -2.0 * p
                psq_smem[c, j] = p * p
            return 0
        jax.lax.fori_loop(0, _NUM_CACHE, body, 0)

    x = x_ref[...]
    x2 = x * x

    inf = jnp.full((1, _LB), jnp.inf, dtype=jnp.float32)
    zero_row = jnp.zeros((1, _LB), dtype=jnp.float32)
    init = (inf, jnp.zeros((1, _LB), dtype=jnp.int32)) + tuple(
        zero_row for _ in range(_P))

    def c_body(c, carry):
        best_mse, best_idx = carry[0], carry[1]
        sps = carry[2:]
        md = None
        for j in range(_P):
            z = x * n2p_smem[c, j] + psq_smem[c, j]
            md = z if md is None else jnp.minimum(md, z)
        md = x2 + md
        mse_c = jnp.sum(md, axis=0, keepdims=True)
        upd = mse_c < best_mse
        best_mse = jnp.where(upd, mse_c, best_mse)
        best_idx = jnp.where(upd, c, best_idx)
        new_sps = tuple(
            jnp.where(upd, p_smem[c, j], sps[j]) for j in range(_P))
        return (best_mse, best_idx) + new_sps

    carry = jax.lax.fori_loop(0, _NUM_CACHE, c_body, init, unroll=4)
    sps = carry[2:]

    q = None
    dbest = None
    for j in range(_P):
        spb = jnp.broadcast_to(sps[j], (_BVR_LEN, _LB))
        t = x - spb
        d = t * t
        if j == 0:
            q, dbest = spb, d
        else:
            upd = d < dbest
            dbest = jnp.where(upd, d, dbest)
            q = jnp.where(upd, spb, q)
    out_ref[...] = q


def kernel(data, coeff_cache):
    orig_shape = data.shape
    flat = data.reshape(-1)
    g = flat.shape[0] // _BVR_LEN
    xt = flat.reshape(g, _BVR_LEN).T

    bin_t = jnp.asarray(_COMBS.astype(np.float32)).T
    points = coeff_cache @ bin_t

    out_t = pl.pallas_call(
        _sbvr_kernel,
        grid=(g // _LB,),
        in_specs=[
            pl.BlockSpec(memory_space=pltpu.SMEM),
            pl.BlockSpec((_BVR_LEN, _LB), lambda i: (0, i)),
        ],
        out_specs=pl.BlockSpec((_BVR_LEN, _LB), lambda i: (0, i)),
        out_shape=jax.ShapeDtypeStruct((_BVR_LEN, g), jnp.float32),
        scratch_shapes=[
            pltpu.SMEM((_NUM_CACHE, _P), jnp.float32),
            pltpu.SMEM((_NUM_CACHE, _P), jnp.float32),
        ],
    )(points, xt)

    return out_t.T.reshape(orig_shape)


# speedup vs baseline: 25.2986x; 1.0128x over previous
import jax
import jax.numpy as jnp
import numpy as np
from jax.experimental import pallas as pl
from jax.experimental.pallas import tpu as pltpu

_NUM_SUMS = 3
_BVR_LEN = 64
_NUM_CACHE = 128
_P = 2 ** _NUM_SUMS
_LB = 128

_COMBS = ((np.arange(_P)[:, None] >> np.arange(_NUM_SUMS - 1, -1, -1)[None, :]) & 1)


def _sbvr_kernel(p_smem, x_ref, out_ref, n2p_smem, psq_smem):
    @pl.when(pl.program_id(0) == 0)
    def _init():
        def body(c, _):
            for j in range(_P):
                p = p_smem[c, j]
                n2p_smem[c, j] = -2.0 * p
                psq_smem[c, j] = p * p
            return 0
        jax.lax.fori_loop(0, _NUM_CACHE, body, 0)

    x = x_ref[...]
    x2 = x * x

    inf = jnp.full((1, _LB), jnp.inf, dtype=jnp.float32)
    zero_row = jnp.zeros((1, _LB), dtype=jnp.float32)
    init = (inf, jnp.zeros((1, _LB), dtype=jnp.int32)) + tuple(
        zero_row for _ in range(_P))

    def c_body(c, carry):
        best_mse, best_idx = carry[0], carry[1]
        sps = carry[2:]
        md = None
        for j in range(_P):
            z = x * n2p_smem[c, j] + psq_smem[c, j]
            md = z if md is None else jnp.minimum(md, z)
        md = x2 + md
        mse_c = jnp.sum(md, axis=0, keepdims=True)
        upd = mse_c < best_mse
        best_mse = jnp.where(upd, mse_c, best_mse)
        best_idx = jnp.where(upd, c, best_idx)
        new_sps = tuple(
            jnp.where(upd, p_smem[c, j], sps[j]) for j in range(_P))
        return (best_mse, best_idx) + new_sps

    carry = jax.lax.fori_loop(0, _NUM_CACHE, c_body, init, unroll=8)
    sps = carry[2:]

    q = None
    dbest = None
    for j in range(_P):
        spb = jnp.broadcast_to(sps[j], (_BVR_LEN, _LB))
        t = x - spb
        d = t * t
        if j == 0:
            q, dbest = spb, d
        else:
            upd = d < dbest
            dbest = jnp.where(upd, d, dbest)
            q = jnp.where(upd, spb, q)
    out_ref[...] = q


def kernel(data, coeff_cache):
    orig_shape = data.shape
    flat = data.reshape(-1)
    g = flat.shape[0] // _BVR_LEN
    xt = flat.reshape(g, _BVR_LEN).T

    bin_t = jnp.asarray(_COMBS.astype(np.float32)).T
    points = coeff_cache @ bin_t

    out_t = pl.pallas_call(
        _sbvr_kernel,
        grid=(g // _LB,),
        in_specs=[
            pl.BlockSpec(memory_space=pltpu.SMEM),
            pl.BlockSpec((_BVR_LEN, _LB), lambda i: (0, i)),
        ],
        out_specs=pl.BlockSpec((_BVR_LEN, _LB), lambda i: (0, i)),
        out_shape=jax.ShapeDtypeStruct((_BVR_LEN, g), jnp.float32),
        scratch_shapes=[
            pltpu.SMEM((_NUM_CACHE, _P), jnp.float32),
            pltpu.SMEM((_NUM_CACHE, _P), jnp.float32),
        ],
    )(points, xt)

    return out_t.T.reshape(orig_shape)


# speedup vs baseline: 28.5269x; 1.1276x over previous
import jax
import jax.numpy as jnp
import numpy as np
from jax.experimental import pallas as pl
from jax.experimental.pallas import tpu as pltpu

_NUM_SUMS = 3
_BVR_LEN = 64
_NUM_CACHE = 128
_P = 2 ** _NUM_SUMS
_LB = 128

_COMBS = ((np.arange(_P)[:, None] >> np.arange(_NUM_SUMS - 1, -1, -1)[None, :]) & 1)


def _sbvr_kernel(p_smem, x_ref, out_ref):
    x = x_ref[...]

    inf = jnp.full((1, _LB), jnp.inf, dtype=jnp.float32)
    zero_row = jnp.zeros((1, _LB), dtype=jnp.float32)
    init = (inf, jnp.zeros((1, _LB), dtype=jnp.int32)) + tuple(
        zero_row for _ in range(_P))

    def c_body(c, carry):
        best_mse, best_idx = carry[0], carry[1]
        sps = carry[2:]
        md = None
        for j in range(_P):
            t = x - p_smem[c, j]
            d = t * t
            md = d if md is None else jnp.minimum(md, d)
        mse_c = jnp.sum(md, axis=0, keepdims=True)
        upd = mse_c < best_mse
        best_mse = jnp.where(upd, mse_c, best_mse)
        best_idx = jnp.where(upd, c, best_idx)
        new_sps = tuple(
            jnp.where(upd, p_smem[c, j], sps[j]) for j in range(_P))
        return (best_mse, best_idx) + new_sps

    carry = jax.lax.fori_loop(0, _NUM_CACHE, c_body, init, unroll=8)
    sps = carry[2:]

    q = None
    dbest = None
    for j in range(_P):
        spb = jnp.broadcast_to(sps[j], (_BVR_LEN, _LB))
        t = x - spb
        d = t * t
        if j == 0:
            q, dbest = spb, d
        else:
            upd = d < dbest
            dbest = jnp.where(upd, d, dbest)
            q = jnp.where(upd, spb, q)
    out_ref[...] = q


def kernel(data, coeff_cache):
    orig_shape = data.shape
    flat = data.reshape(-1)
    g = flat.shape[0] // _BVR_LEN
    xt = flat.reshape(g, _BVR_LEN).T

    bin_t = jnp.asarray(_COMBS.astype(np.float32)).T
    points = coeff_cache @ bin_t

    out_t = pl.pallas_call(
        _sbvr_kernel,
        grid=(g // _LB,),
        in_specs=[
            pl.BlockSpec(memory_space=pltpu.SMEM),
            pl.BlockSpec((_BVR_LEN, _LB), lambda i: (0, i)),
        ],
        out_specs=pl.BlockSpec((_BVR_LEN, _LB), lambda i: (0, i)),
        out_shape=jax.ShapeDtypeStruct((_BVR_LEN, g), jnp.float32),
    )(points, xt)

    return out_t.T.reshape(orig_shape)


# speedup vs baseline: 35.7789x; 1.2542x over previous
import functools

import jax
import jax.numpy as jnp
import numpy as np
from jax import lax
from jax.experimental import pallas as pl
from jax.experimental.pallas import tpu as pltpu
from jax.experimental.pallas import tpu_sc as plsc

_NUM_SUMS = 3
_BVR_LEN = 64
_NUM_CACHE = 128
_P = 2 ** _NUM_SUMS
_LB = 128

_COMBS = ((np.arange(_P)[:, None] >> np.arange(_NUM_SUMS - 1, -1, -1)[None, :]) & 1)


def _sbvr_kernel(p_smem, x_ref, out_ref):
    x = x_ref[...]

    inf = jnp.full((1, _LB), jnp.inf, dtype=jnp.float32)
    zero_row = jnp.zeros((1, _LB), dtype=jnp.float32)
    init = (inf, jnp.zeros((1, _LB), dtype=jnp.int32)) + tuple(
        zero_row for _ in range(_P))

    def c_body(c, carry):
        best_mse, best_idx = carry[0], carry[1]
        sps = carry[2:]
        md = None
        for j in range(_P):
            t = x - p_smem[c, j]
            d = t * t
            md = d if md is None else jnp.minimum(md, d)
        mse_c = jnp.sum(md, axis=0, keepdims=True)
        upd = mse_c < best_mse
        best_mse = jnp.where(upd, mse_c, best_mse)
        best_idx = jnp.where(upd, c, best_idx)
        new_sps = tuple(
            jnp.where(upd, p_smem[c, j], sps[j]) for j in range(_P))
        return (best_mse, best_idx) + new_sps

    carry = jax.lax.fori_loop(0, _NUM_CACHE, c_body, init, unroll=8)
    sps = carry[2:]

    q = None
    dbest = None
    for j in range(_P):
        spb = jnp.broadcast_to(sps[j], (_BVR_LEN, _LB))
        t = x - spb
        d = t * t
        if j == 0:
            q, dbest = spb, d
        else:
            upd = d < dbest
            dbest = jnp.where(upd, d, dbest)
            q = jnp.where(upd, spb, q)
    out_ref[...] = q


_G_SC = 4096
_NW = 32
_GPW = _G_SC // _NW
_NT = _GPW // 16


def _sc_sbvr(points_flat, xt_sc):
    mesh = plsc.VectorSubcoreMesh(core_axis_name="c", subcore_axis_name="s")
    cp = pltpu.CompilerParams()
    if "needs_layout_passes" in pltpu.CompilerParams.__dataclass_fields__:
        import dataclasses
        cp = dataclasses.replace(cp, needs_layout_passes=False)

    @functools.partial(
        pl.kernel, mesh=mesh, compiler_params=cp,
        out_type=jax.ShapeDtypeStruct((_BVR_LEN, _G_SC), jnp.float32),
        scratch_types=[
            pltpu.VMEM((_NUM_CACHE * _P,), jnp.float32),
            pltpu.VMEM((_BVR_LEN, _GPW), jnp.float32),
            pltpu.VMEM((_BVR_LEN, _GPW), jnp.float32),
            pltpu.SemaphoreType.DMA,
        ],
    )
    def k(pts_hbm, x_hbm, o_hbm, pts_v, x_v, o_v, sem):
        wid = lax.axis_index("s") * 2 + lax.axis_index("c")
        base = wid * _GPW
        pltpu.async_copy(pts_hbm, pts_v, sem).wait()
        pltpu.async_copy(x_hbm.at[:, pl.ds(base, _GPW)], x_v, sem).wait()

        def splat_gather(idx_vec):
            return plsc.load_gather(pts_v, [idx_vec])

        def tile_body(t, _):
            off = t * 16

            def c_body(c, carry):
                bm, bi = carry
                pvs = [splat_gather(jnp.broadcast_to(c * _P + j, (16,)))
                       for j in range(_P)]

                def l_body(i, accs):
                    new = []
                    for u in range(4):
                        xl = x_v[u * 16 + i, pl.ds(off, 16)]
                        d = None
                        for j in range(_P):
                            tj = xl - pvs[j]
                            dj = tj * tj
                            d = dj if d is None else jnp.minimum(d, dj)
                        new.append(accs[u] + d)
                    return tuple(new)

                z = jnp.zeros((16,), jnp.float32)
                a0, a1, a2, a3 = lax.fori_loop(0, 16, l_body, (z, z, z, z))
                mse = (a0 + a1) + (a2 + a3)
                upd = mse < bm
                bm = jnp.where(upd, mse, bm)
                bi = jnp.where(upd, jnp.broadcast_to(c, (16,)), bi)
                return bm, bi

            init = (jnp.full((16,), jnp.inf, jnp.float32),
                    jnp.zeros((16,), jnp.int32))
            _, bi = lax.fori_loop(0, _NUM_CACHE, c_body, init)

            psel = [splat_gather(bi * _P + j) for j in range(_P)]

            def q_body(l, _):
                xl = x_v[l, pl.ds(off, 16)]
                q = None
                db = None
                for j in range(_P):
                    tj = xl - psel[j]
                    dj = tj * tj
                    if j == 0:
                        q, db = psel[j], dj
                    else:
                        u2 = dj < db
                        db = jnp.where(u2, dj, db)
                        q = jnp.where(u2, psel[j], q)
                o_v[l, pl.ds(off, 16)] = q
                return 0

            lax.fori_loop(0, _BVR_LEN, q_body, 0)
            return 0

        lax.fori_loop(0, _NT, tile_body, 0)
        pltpu.async_copy(o_v, o_hbm.at[:, pl.ds(base, _GPW)], sem).wait()

    return k(points_flat, xt_sc)


def kernel(data, coeff_cache):
    orig_shape = data.shape
    flat = data.reshape(-1)
    g = flat.shape[0] // _BVR_LEN
    xt = flat.reshape(g, _BVR_LEN).T

    bin_t = jnp.asarray(_COMBS.astype(np.float32)).T
    points = coeff_cache @ bin_t

    g_tc = g - _G_SC
    out_tc = pl.pallas_call(
        _sbvr_kernel,
        grid=(g_tc // _LB,),
        in_specs=[
            pl.BlockSpec(memory_space=pltpu.SMEM),
            pl.BlockSpec((_BVR_LEN, _LB), lambda i: (0, i)),
        ],
        out_specs=pl.BlockSpec((_BVR_LEN, _LB), lambda i: (0, i)),
        out_shape=jax.ShapeDtypeStruct((_BVR_LEN, g_tc), jnp.float32),
    )(points, xt[:, :g_tc])

    out_sc = _sc_sbvr(points.reshape(-1), xt[:, g_tc:])

    out_t = jnp.concatenate([out_tc, out_sc], axis=1)
    return out_t.T.reshape(orig_shape)


# speedup vs baseline: 36.2631x; 1.0135x over previous
import functools

import jax
import jax.numpy as jnp
import numpy as np
from jax import lax
from jax.experimental import pallas as pl
from jax.experimental.pallas import tpu as pltpu
from jax.experimental.pallas import tpu_sc as plsc

_NUM_SUMS = 3
_BVR_LEN = 64
_NUM_CACHE = 128
_P = 2 ** _NUM_SUMS
_LB = 256

_COMBS = ((np.arange(_P)[:, None] >> np.arange(_NUM_SUMS - 1, -1, -1)[None, :]) & 1)


def _sbvr_kernel(p_smem, x_ref, out_ref):
    x = x_ref[...]

    inf = jnp.full((1, _LB), jnp.inf, dtype=jnp.float32)
    zero_row = jnp.zeros((1, _LB), dtype=jnp.float32)
    init = (inf, jnp.zeros((1, _LB), dtype=jnp.int32)) + tuple(
        zero_row for _ in range(_P))

    def c_body(c, carry):
        best_mse, best_idx = carry[0], carry[1]
        sps = carry[2:]
        md = None
        for j in range(_P):
            t = x - p_smem[c, j]
            d = t * t
            md = d if md is None else jnp.minimum(md, d)
        mse_c = jnp.sum(md, axis=0, keepdims=True)
        upd = mse_c < best_mse
        best_mse = jnp.where(upd, mse_c, best_mse)
        best_idx = jnp.where(upd, c, best_idx)
        new_sps = tuple(
            jnp.where(upd, p_smem[c, j], sps[j]) for j in range(_P))
        return (best_mse, best_idx) + new_sps

    carry = jax.lax.fori_loop(0, _NUM_CACHE, c_body, init, unroll=4)
    sps = carry[2:]

    q = None
    dbest = None
    for j in range(_P):
        spb = jnp.broadcast_to(sps[j], (_BVR_LEN, _LB))
        t = x - spb
        d = t * t
        if j == 0:
            q, dbest = spb, d
        else:
            upd = d < dbest
            dbest = jnp.where(upd, d, dbest)
            q = jnp.where(upd, spb, q)
    out_ref[...] = q


_G_SC = 4096
_NW = 32
_GPW = _G_SC // _NW
_NT = _GPW // 16


def _sc_sbvr(points_flat, xt_sc):
    mesh = plsc.VectorSubcoreMesh(core_axis_name="c", subcore_axis_name="s")
    cp = pltpu.CompilerParams()
    if "needs_layout_passes" in pltpu.CompilerParams.__dataclass_fields__:
        import dataclasses
        cp = dataclasses.replace(cp, needs_layout_passes=False)

    @functools.partial(
        pl.kernel, mesh=mesh, compiler_params=cp,
        out_type=jax.ShapeDtypeStruct((_BVR_LEN, _G_SC), jnp.float32),
        scratch_types=[
            pltpu.VMEM((_NUM_CACHE * _P,), jnp.float32),
            pltpu.VMEM((_BVR_LEN, _GPW), jnp.float32),
            pltpu.VMEM((_BVR_LEN, _GPW), jnp.float32),
            pltpu.SemaphoreType.DMA,
        ],
    )
    def k(pts_hbm, x_hbm, o_hbm, pts_v, x_v, o_v, sem):
        wid = lax.axis_index("s") * 2 + lax.axis_index("c")
        base = wid * _GPW
        pltpu.async_copy(pts_hbm, pts_v, sem).wait()
        pltpu.async_copy(x_hbm.at[:, pl.ds(base, _GPW)], x_v, sem).wait()

        def splat_gather(idx_vec):
            return plsc.load_gather(pts_v, [idx_vec])

        def tile_body(t, _):
            off = t * 16

            def c_body(c, carry):
                bm, bi = carry
                pvs = [splat_gather(jnp.broadcast_to(c * _P + j, (16,)))
                       for j in range(_P)]

                def l_body(i, accs):
                    new = []
                    for u in range(4):
                        xl = x_v[u * 16 + i, pl.ds(off, 16)]
                        d = None
                        for j in range(_P):
                            tj = xl - pvs[j]
                            dj = tj * tj
                            d = dj if d is None else jnp.minimum(d, dj)
                        new.append(accs[u] + d)
                    return tuple(new)

                z = jnp.zeros((16,), jnp.float32)
                a0, a1, a2, a3 = lax.fori_loop(0, 16, l_body, (z, z, z, z))
                mse = (a0 + a1) + (a2 + a3)
                upd = mse < bm
                bm = jnp.where(upd, mse, bm)
                bi = jnp.where(upd, jnp.broadcast_to(c, (16,)), bi)
                return bm, bi

            init = (jnp.full((16,), jnp.inf, jnp.float32),
                    jnp.zeros((16,), jnp.int32))
            _, bi = lax.fori_loop(0, _NUM_CACHE, c_body, init)

            psel = [splat_gather(bi * _P + j) for j in range(_P)]

            def q_body(l, _):
                xl = x_v[l, pl.ds(off, 16)]
                q = None
                db = None
                for j in range(_P):
                    tj = xl - psel[j]
                    dj = tj * tj
                    if j == 0:
                        q, db = psel[j], dj
                    else:
                        u2 = dj < db
                        db = jnp.where(u2, dj, db)
                        q = jnp.where(u2, psel[j], q)
                o_v[l, pl.ds(off, 16)] = q
                return 0

            lax.fori_loop(0, _BVR_LEN, q_body, 0)
            return 0

        lax.fori_loop(0, _NT, tile_body, 0)
        pltpu.async_copy(o_v, o_hbm.at[:, pl.ds(base, _GPW)], sem).wait()

    return k(points_flat, xt_sc)


def kernel(data, coeff_cache):
    orig_shape = data.shape
    flat = data.reshape(-1)
    g = flat.shape[0] // _BVR_LEN
    xt = flat.reshape(g, _BVR_LEN).T

    bin_t = jnp.asarray(_COMBS.astype(np.float32)).T
    points = coeff_cache @ bin_t

    g_tc = g - _G_SC
    out_tc = pl.pallas_call(
        _sbvr_kernel,
        grid=(g_tc // _LB,),
        in_specs=[
            pl.BlockSpec(memory_space=pltpu.SMEM),
            pl.BlockSpec((_BVR_LEN, _LB), lambda i: (0, i)),
        ],
        out_specs=pl.BlockSpec((_BVR_LEN, _LB), lambda i: (0, i)),
        out_shape=jax.ShapeDtypeStruct((_BVR_LEN, g_tc), jnp.float32),
    )(points, xt[:, :g_tc])

    out_sc = _sc_sbvr(points.reshape(-1), xt[:, g_tc:])

    out_t = jnp.concatenate([out_tc, out_sc], axis=1)
    return out_t.T.reshape(orig_shape)
